# baseline (speedup 1.0000x reference)
import functools

import jax
import jax.numpy as jnp
from jax import lax
from jax.experimental import pallas as pl
from jax.experimental.pallas import tpu as pltpu
from jax.experimental.pallas import tpu_sc as plsc

NC = 2
NS = 16
L = 16
NW = NC * NS

BATCH = 4
SEQ = 2048
D = 1024
S_PER_W = SEQ // NW
PE_WORDS = S_PER_W * D
CH_ROWS = 16
CH_WORDS = CH_ROWS * D
N_CH = S_PER_W // CH_ROWS


def _sc_body(x_hbm, pe_hbm, out_hbm, peb, xb):
    w = lax.axis_index("s") * NC + lax.axis_index("c")
    pe_off = w * PE_WORDS
    pltpu.sync_copy(pe_hbm.at[pl.ds(pe_off, PE_WORDS)], peb)

    def add_chunk(k_base):
        def body(i, carry):
            s = i * L
            xb[pl.ds(s, L)] = xb[pl.ds(s, L)] + peb[pl.ds(k_base + s, L)]
            return carry
        lax.fori_loop(0, CH_WORDS // L, body, 0, unroll=8)

    for b in range(BATCH):
        for k in range(N_CH):
            off = b * (SEQ * D) + pe_off + k * CH_WORDS
            pltpu.sync_copy(x_hbm.at[pl.ds(off, CH_WORDS)], xb)
            add_chunk(k * CH_WORDS)
            pltpu.sync_copy(xb, out_hbm.at[pl.ds(off, CH_WORDS)])


@functools.partial(jax.jit, static_argnums=())
def _sc_call(x_flat, pe_flat):
    mesh = plsc.VectorSubcoreMesh(core_axis_name="c", subcore_axis_name="s")
    return pl.kernel(
        _sc_body,
        out_type=jax.ShapeDtypeStruct((BATCH * SEQ * D,), jnp.float32),
        mesh=mesh,
        scratch_types=[
            pltpu.VMEM((PE_WORDS,), jnp.float32),
            pltpu.VMEM((CH_WORDS,), jnp.float32),
        ],
    )(x_flat, pe_flat)


def kernel(x, pos_table):
    batch, seq_len, d_model = x.shape
    pe = pos_table[:seq_len]
    out = _sc_call(x.reshape(-1), pe.reshape(-1))
    return out.reshape(x.shape)


# speedup vs baseline: 1.3249x; 1.3249x over previous
import functools

import jax
import jax.numpy as jnp
from jax import lax
from jax.experimental import pallas as pl
from jax.experimental.pallas import tpu as pltpu
from jax.experimental.pallas import tpu_sc as plsc

NC = 2
NS = 16
L = 16
NW = NC * NS

BATCH = 4
SEQ = 2048
D = 1024
S_PER_W = SEQ // NW
PE_WORDS = S_PER_W * D
CH_ROWS = 16
CH_WORDS = CH_ROWS * D
N_CH = S_PER_W // CH_ROWS
NIT = BATCH * N_CH
DEPTH = 3


def _sc_body(x_hbm, pe_hbm, out_hbm,
             peb, xb0, xb1, xb2,
             pes, si0, si1, si2, so0, so1, so2):
    w = lax.axis_index("s") * NC + lax.axis_index("c")
    pe_off = w * PE_WORDS
    bufs = [xb0, xb1, xb2]
    isems = [si0, si1, si2]
    osems = [so0, so1, so2]

    def off_of(t):
        b, k = divmod(t, N_CH)
        return b * (SEQ * D) + pe_off + k * CH_WORDS, k

    pe_cp = pltpu.async_copy(pe_hbm.at[pl.ds(pe_off, PE_WORDS)], peb, pes)

    in_h = {}
    out_h = {}
    for t in range(2):
        off, _ = off_of(t)
        in_h[t] = pltpu.async_copy(
            x_hbm.at[pl.ds(off, CH_WORDS)], bufs[t % DEPTH], isems[t % DEPTH])
    pe_cp.wait()

    for t in range(NIT):
        j = t % DEPTH
        off, k = off_of(t)
        xb = bufs[j]
        in_h[t].wait()
        k_base = k * CH_WORDS

        @plsc.parallel_loop(0, CH_WORDS, step=L, unroll=8)
        def _add(i):
            xb[pl.ds(i, L)] = xb[pl.ds(i, L)] + peb[pl.ds(k_base + i, L)]

        out_h[t] = pltpu.async_copy(xb, out_hbm.at[pl.ds(off, CH_WORDS)], osems[j])
        nt = t + 2
        if nt < NIT:
            noff, _ = off_of(nt)
            if t >= 1:
                out_h[t - 1].wait()
            in_h[nt] = pltpu.async_copy(
                x_hbm.at[pl.ds(noff, CH_WORDS)], bufs[nt % DEPTH], isems[nt % DEPTH])

    out_h[NIT - 2].wait()
    out_h[NIT - 1].wait()


@jax.jit
def _sc_call(x_flat, pe_flat):
    mesh = plsc.VectorSubcoreMesh(core_axis_name="c", subcore_axis_name="s")
    return pl.kernel(
        _sc_body,
        out_type=jax.ShapeDtypeStruct((BATCH * SEQ * D,), jnp.float32),
        mesh=mesh,
        scratch_types=[
            pltpu.VMEM((PE_WORDS,), jnp.float32),
            pltpu.VMEM((CH_WORDS,), jnp.float32),
            pltpu.VMEM((CH_WORDS,), jnp.float32),
            pltpu.VMEM((CH_WORDS,), jnp.float32),
            pltpu.SemaphoreType.DMA,
            pltpu.SemaphoreType.DMA,
            pltpu.SemaphoreType.DMA,
            pltpu.SemaphoreType.DMA,
            pltpu.SemaphoreType.DMA,
            pltpu.SemaphoreType.DMA,
            pltpu.SemaphoreType.DMA,
        ],
    )(x_flat, pe_flat)


def kernel(x, pos_table):
    batch, seq_len, d_model = x.shape
    pe = pos_table[:seq_len]
    out = _sc_call(x.reshape(-1), pe.reshape(-1))
    return out.reshape(x.shape)


# speedup vs baseline: 3.1745x; 2.3960x over previous
import jax
import jax.numpy as jnp
from jax import lax
from jax.experimental import pallas as pl
from jax.experimental.pallas import tpu as pltpu
from jax.experimental.pallas import tpu_sc as plsc

NC = 2
NS = 16
L = 16
NW = NC * NS

BATCH = 4
SEQ = 2048
D = 1024
S_PER_W = SEQ // NW
CH_ROWS = 16
N_CH = S_PER_W // CH_ROWS
NIT = BATCH * N_CH
DEPTH = 3


def _sc_body(x_hbm, pe_hbm, out_hbm,
             peb, xb0, xb1, xb2,
             pes, si0, si1, si2, so0, so1, so2):
    w = lax.axis_index("s") * NC + lax.axis_index("c")
    s0 = w * S_PER_W
    bufs = [xb0, xb1, xb2]
    isems = [si0, si1, si2]
    osems = [so0, so1, so2]

    pe_cp = pltpu.async_copy(pe_hbm.at[pl.ds(s0, S_PER_W)], peb, pes)

    def chunk_of(t):
        b, k = divmod(t, N_CH)
        return b, k

    in_h = {}
    out_h = {}
    for t in range(2):
        b, k = chunk_of(t)
        in_h[t] = pltpu.async_copy(
            x_hbm.at[b, pl.ds(s0 + k * CH_ROWS, CH_ROWS)],
            bufs[t % DEPTH], isems[t % DEPTH])
    pe_cp.wait()

    for t in range(NIT):
        j = t % DEPTH
        b, k = chunk_of(t)
        xb = bufs[j]
        in_h[t].wait()

        def row_body(r, carry, _xb=xb, _k=k):
            pr = _k * CH_ROWS + r

            @plsc.parallel_loop(0, D, step=L, unroll=8)
            def _add(i):
                _xb[r, pl.ds(i, L)] = _xb[r, pl.ds(i, L)] + peb[pr, pl.ds(i, L)]

            return carry

        lax.fori_loop(0, CH_ROWS, row_body, 0)

        out_h[t] = pltpu.async_copy(
            xb, out_hbm.at[b, pl.ds(s0 + k * CH_ROWS, CH_ROWS)], osems[j])
        nt = t + 2
        if nt < NIT:
            nb, nk = chunk_of(nt)
            if t >= 1:
                out_h[t - 1].wait()
            in_h[nt] = pltpu.async_copy(
                x_hbm.at[nb, pl.ds(s0 + nk * CH_ROWS, CH_ROWS)],
                bufs[nt % DEPTH], isems[nt % DEPTH])

    out_h[NIT - 2].wait()
    out_h[NIT - 1].wait()


@jax.jit
def _sc_call(x, pe):
    mesh = plsc.VectorSubcoreMesh(core_axis_name="c", subcore_axis_name="s")
    return pl.kernel(
        _sc_body,
        out_type=jax.ShapeDtypeStruct((BATCH, SEQ, D), jnp.float32),
        mesh=mesh,
        scratch_types=[
            pltpu.VMEM((S_PER_W, D), jnp.float32),
            pltpu.VMEM((CH_ROWS, D), jnp.float32),
            pltpu.VMEM((CH_ROWS, D), jnp.float32),
            pltpu.VMEM((CH_ROWS, D), jnp.float32),
            pltpu.SemaphoreType.DMA,
            pltpu.SemaphoreType.DMA,
            pltpu.SemaphoreType.DMA,
            pltpu.SemaphoreType.DMA,
            pltpu.SemaphoreType.DMA,
            pltpu.SemaphoreType.DMA,
            pltpu.SemaphoreType.DMA,
        ],
    )(x, pe)


def kernel(x, pos_table):
    batch, seq_len, d_model = x.shape
    pe = pos_table[:seq_len]
    return _sc_call(x, pe)


# speedup vs baseline: 3.1791x; 1.0014x over previous
import jax
import jax.numpy as jnp
from jax import lax
from jax.experimental import pallas as pl
from jax.experimental.pallas import tpu as pltpu
from jax.experimental.pallas import tpu_sc as plsc

NC = 2
NS = 16
L = 16
NW = NC * NS

BATCH = 4
SEQ = 2048
D = 1024
S_PER_W = SEQ // NW
CH_ROWS = 4
N_CH = S_PER_W // CH_ROWS
DEPTH = 3


def _sc_body(x_hbm, pe_hbm, out_hbm,
             peb, xb0, xb1, xb2,
             pes, si0, si1, si2, so0, so1, so2):
    w = lax.axis_index("s") * NC + lax.axis_index("c")
    s0 = w * S_PER_W
    bufs = [xb0, xb1, xb2]
    isems = [si0, si1, si2]
    osems = [so0, so1, so2]

    pe_cp = pltpu.async_copy(pe_hbm.at[pl.ds(s0, S_PER_W)], peb, pes)

    def start_in(t):
        hs = []
        for b in range(BATCH):
            hs.append(pltpu.async_copy(
                x_hbm.at[b, pl.ds(s0 + t * CH_ROWS, CH_ROWS)],
                bufs[t % DEPTH].at[b], isems[t % DEPTH]))
        return hs

    def start_out(t):
        hs = []
        for b in range(BATCH):
            hs.append(pltpu.async_copy(
                bufs[t % DEPTH].at[b],
                out_hbm.at[b, pl.ds(s0 + t * CH_ROWS, CH_ROWS)],
                osems[t % DEPTH]))
        return hs

    in_h = {}
    out_h = {}
    for t in range(2):
        in_h[t] = start_in(t)
    pe_cp.wait()

    for t in range(N_CH):
        xb = bufs[t % DEPTH]
        for h in in_h[t]:
            h.wait()

        def row_body(r, carry, _xb=xb, _t=t):
            pr = _t * CH_ROWS + r

            @plsc.parallel_loop(0, D, step=L, unroll=8)
            def _add(i):
                pe_v = peb[pr, pl.ds(i, L)]
                for b in range(BATCH):
                    _xb[b, r, pl.ds(i, L)] = _xb[b, r, pl.ds(i, L)] + pe_v

            return carry

        lax.fori_loop(0, CH_ROWS, row_body, 0)

        out_h[t] = start_out(t)
        nt = t + 2
        if nt < N_CH:
            if t >= 1:
                for h in out_h[t - 1]:
                    h.wait()
            in_h[nt] = start_in(nt)

    for t in (N_CH - 2, N_CH - 1):
        for h in out_h[t]:
            h.wait()


@jax.jit
def _sc_call(x, pe):
    mesh = plsc.VectorSubcoreMesh(core_axis_name="c", subcore_axis_name="s")
    return pl.kernel(
        _sc_body,
        out_type=jax.ShapeDtypeStruct((BATCH, SEQ, D), jnp.float32),
        mesh=mesh,
        scratch_types=[
            pltpu.VMEM((S_PER_W, D), jnp.float32),
            pltpu.VMEM((BATCH, CH_ROWS, D), jnp.float32),
            pltpu.VMEM((BATCH, CH_ROWS, D), jnp.float32),
            pltpu.VMEM((BATCH, CH_ROWS, D), jnp.float32),
            pltpu.SemaphoreType.DMA,
            pltpu.SemaphoreType.DMA,
            pltpu.SemaphoreType.DMA,
            pltpu.SemaphoreType.DMA,
            pltpu.SemaphoreType.DMA,
            pltpu.SemaphoreType.DMA,
            pltpu.SemaphoreType.DMA,
        ],
    )(x, pe)


def kernel(x, pos_table):
    batch, seq_len, d_model = x.shape
    pe = pos_table[:seq_len]
    return _sc_call(x, pe)


# speedup vs baseline: 3.2684x; 1.0281x over previous
import jax
import jax.numpy as jnp
from jax import lax
from jax.experimental import pallas as pl
from jax.experimental.pallas import tpu as pltpu
from jax.experimental.pallas import tpu_sc as plsc

NC = 2
NS = 16
L = 16
NW = NC * NS

BATCH = 4
SEQ = 2048
D = 1024
S_PER_W = SEQ // NW
CH_ROWS = 8
K_CH = S_PER_W // CH_ROWS
NIT = BATCH * K_CH
DEPTH = 6
PE_PIECES = 4
PE_PIECE_ROWS = S_PER_W // PE_PIECES


def _sc_body(x_hbm, pe_hbm, out_hbm, peb, *rest):
    bufs = list(rest[:DEPTH])
    pes = rest[DEPTH]
    isems = list(rest[DEPTH + 1:DEPTH + 1 + DEPTH])
    osems = list(rest[DEPTH + 1 + DEPTH:DEPTH + 1 + 2 * DEPTH])

    w = lax.axis_index("s") * NC + lax.axis_index("c")
    s0 = w * S_PER_W

    pe_h = []
    for p in range(PE_PIECES):
        pe_h.append(pltpu.async_copy(
            pe_hbm.at[pl.ds(s0 + p * PE_PIECE_ROWS, PE_PIECE_ROWS)],
            peb.at[pl.ds(p * PE_PIECE_ROWS, PE_PIECE_ROWS)], pes))

    def chunk_of(t):
        b, k = divmod(t, K_CH)
        return b, k

    def start_in(t):
        b, k = chunk_of(t)
        return pltpu.async_copy(
            x_hbm.at[b, pl.ds(s0 + k * CH_ROWS, CH_ROWS)],
            bufs[t % DEPTH], isems[t % DEPTH])

    def start_out(t):
        b, k = chunk_of(t)
        return pltpu.async_copy(
            bufs[t % DEPTH],
            out_hbm.at[b, pl.ds(s0 + k * CH_ROWS, CH_ROWS)], osems[t % DEPTH])

    AHEAD = 3
    in_h = {}
    out_h = {}
    for t in range(AHEAD):
        in_h[t] = start_in(t)

    pe_waited = 0
    for t in range(NIT):
        b, k = chunk_of(t)
        xb = bufs[t % DEPTH]
        need_piece = min(k // (PE_PIECE_ROWS // CH_ROWS) + 1, PE_PIECES)
        while pe_waited < need_piece:
            pe_h[pe_waited].wait()
            pe_waited += 1
        in_h[t].wait()

        def row_body(r, carry, _xb=xb, _k=k):
            pr = _k * CH_ROWS + r

            @plsc.parallel_loop(0, D, step=L, unroll=8)
            def _add(i):
                _xb[r, pl.ds(i, L)] = _xb[r, pl.ds(i, L)] + peb[pr, pl.ds(i, L)]

            return carry

        lax.fori_loop(0, CH_ROWS, row_body, 0)

        out_h[t] = start_out(t)
        nt = t + AHEAD
        if nt < NIT:
            prev = nt - DEPTH
            if prev >= 0:
                out_h[prev].wait()
            in_h[nt] = start_in(nt)

    for t in range(max(0, NIT - DEPTH), NIT):
        if t in out_h:
            out_h[t].wait()


@jax.jit
def _sc_call(x, pe):
    mesh = plsc.VectorSubcoreMesh(core_axis_name="c", subcore_axis_name="s")
    return pl.kernel(
        _sc_body,
        out_type=jax.ShapeDtypeStruct((BATCH, SEQ, D), jnp.float32),
        mesh=mesh,
        scratch_types=(
            [pltpu.VMEM((S_PER_W, D), jnp.float32)]
            + [pltpu.VMEM((CH_ROWS, D), jnp.float32) for _ in range(DEPTH)]
            + [pltpu.SemaphoreType.DMA for _ in range(1 + 2 * DEPTH)]
        ),
    )(x, pe)


def kernel(x, pos_table):
    batch, seq_len, d_model = x.shape
    pe = pos_table[:seq_len]
    return _sc_call(x, pe)


# speedup vs baseline: 3.3775x; 1.0334x over previous
import jax
import jax.numpy as jnp
from jax import lax
from jax.experimental import pallas as pl
from jax.experimental.pallas import tpu as pltpu
from jax.experimental.pallas import tpu_sc as plsc

NC = 2
NS = 16
L = 16
NW = NC * NS

BATCH = 4
SEQ = 2048
D = 1024
S_PER_W = SEQ // NW
CH_ROWS = 8
K_CH = S_PER_W // CH_ROWS
NIT = BATCH * K_CH
DEPTH = 6
PE_PIECES = 4
PE_PIECE_ROWS = S_PER_W // PE_PIECES


def _sc_body(x_hbm, pe_hbm, out_hbm, peb, *rest):
    bufs = list(rest[:DEPTH])
    pes = rest[DEPTH]
    isems = list(rest[DEPTH + 1:DEPTH + 1 + DEPTH])
    osems = list(rest[DEPTH + 1 + DEPTH:DEPTH + 1 + 2 * DEPTH])

    w = lax.axis_index("s") * NC + lax.axis_index("c")
    s0 = w * S_PER_W

    pe_h = []
    for p in range(PE_PIECES):
        pe_h.append(pltpu.async_copy(
            pe_hbm.at[pl.ds(s0 + p * PE_PIECE_ROWS, PE_PIECE_ROWS)],
            peb.at[pl.ds(p * PE_PIECE_ROWS, PE_PIECE_ROWS)], pes))

    def chunk_of(t):
        b, k = divmod(t, K_CH)
        return b, k

    def start_in(t):
        b, k = chunk_of(t)
        return pltpu.async_copy(
            x_hbm.at[b, pl.ds(s0 + k * CH_ROWS, CH_ROWS)],
            bufs[t % DEPTH], isems[t % DEPTH])

    def start_out(t):
        b, k = chunk_of(t)
        return pltpu.async_copy(
            bufs[t % DEPTH],
            out_hbm.at[b, pl.ds(s0 + k * CH_ROWS, CH_ROWS)], osems[t % DEPTH])

    AHEAD = 4
    in_h = {}
    out_h = {}
    for t in range(AHEAD):
        in_h[t] = start_in(t)

    pe_waited = 0
    for t in range(NIT):
        b, k = chunk_of(t)
        xb = bufs[t % DEPTH]
        need_piece = min(k // (PE_PIECE_ROWS // CH_ROWS) + 1, PE_PIECES)
        while pe_waited < need_piece:
            pe_h[pe_waited].wait()
            pe_waited += 1
        in_h[t].wait()

        def row_body(r, carry, _xb=xb, _k=k):
            pr = _k * CH_ROWS + r

            @plsc.parallel_loop(0, D, step=L, unroll=8)
            def _add(i):
                _xb[r, pl.ds(i, L)] = _xb[r, pl.ds(i, L)] + peb[pr, pl.ds(i, L)]

            return carry

        lax.fori_loop(0, CH_ROWS, row_body, 0)

        out_h[t] = start_out(t)
        nt = t + AHEAD
        if nt < NIT:
            prev = nt - DEPTH
            if prev >= 0:
                out_h[prev].wait()
            in_h[nt] = start_in(nt)

    for t in range(max(0, NIT - DEPTH), NIT):
        if t in out_h:
            out_h[t].wait()


@jax.jit
def _sc_call(x, pe):
    mesh = plsc.VectorSubcoreMesh(core_axis_name="c", subcore_axis_name="s")
    return pl.kernel(
        _sc_body,
        out_type=jax.ShapeDtypeStruct((BATCH, SEQ, D), jnp.float32),
        mesh=mesh,
        scratch_types=(
            [pltpu.VMEM((S_PER_W, D), jnp.float32)]
            + [pltpu.VMEM((CH_ROWS, D), jnp.float32) for _ in range(DEPTH)]
            + [pltpu.SemaphoreType.DMA for _ in range(1 + 2 * DEPTH)]
        ),
    )(x, pe)


def kernel(x, pos_table):
    batch, seq_len, d_model = x.shape
    pe = pos_table[:seq_len]
    return _sc_call(x, pe)


# speedup vs baseline: 3.3842x; 1.0020x over previous
import jax
import jax.numpy as jnp
from jax import lax
from jax.experimental import pallas as pl
from jax.experimental.pallas import tpu as pltpu
from jax.experimental.pallas import tpu_sc as plsc

NC = 2
NS = 16
L = 16
NW = NC * NS

BATCH = 4
SEQ = 2048
D = 1024
S_PER_W = SEQ // NW
CH_ROWS = 8
K_CH = S_PER_W // CH_ROWS
NIT = BATCH * K_CH
DEPTH = 7
PE_PIECES = 4
PE_PIECE_ROWS = S_PER_W // PE_PIECES


def _sc_body(x_hbm, pe_hbm, out_hbm, peb, *rest):
    bufs = list(rest[:DEPTH])
    pes = rest[DEPTH]
    isems = list(rest[DEPTH + 1:DEPTH + 1 + DEPTH])
    osems = list(rest[DEPTH + 1 + DEPTH:DEPTH + 1 + 2 * DEPTH])

    w = lax.axis_index("s") * NC + lax.axis_index("c")
    s0 = w * S_PER_W

    pe_h = []
    for p in range(PE_PIECES):
        pe_h.append(pltpu.async_copy(
            pe_hbm.at[pl.ds(s0 + p * PE_PIECE_ROWS, PE_PIECE_ROWS)],
            peb.at[pl.ds(p * PE_PIECE_ROWS, PE_PIECE_ROWS)], pes))

    def chunk_of(t):
        b, k = divmod(t, K_CH)
        return b, k

    def start_in(t):
        b, k = chunk_of(t)
        return pltpu.async_copy(
            x_hbm.at[b, pl.ds(s0 + k * CH_ROWS, CH_ROWS)],
            bufs[t % DEPTH], isems[t % DEPTH])

    def start_out(t):
        b, k = chunk_of(t)
        return pltpu.async_copy(
            bufs[t % DEPTH],
            out_hbm.at[b, pl.ds(s0 + k * CH_ROWS, CH_ROWS)], osems[t % DEPTH])

    AHEAD = 5
    in_h = {}
    out_h = {}
    for t in range(AHEAD):
        in_h[t] = start_in(t)

    pe_waited = 0
    for t in range(NIT):
        b, k = chunk_of(t)
        xb = bufs[t % DEPTH]
        need_piece = min(k // (PE_PIECE_ROWS // CH_ROWS) + 1, PE_PIECES)
        while pe_waited < need_piece:
            pe_h[pe_waited].wait()
            pe_waited += 1
        in_h[t].wait()

        def row_body(r, carry, _xb=xb, _k=k):
            pr = _k * CH_ROWS + r

            @plsc.parallel_loop(0, D, step=L, unroll=8)
            def _add(i):
                _xb[r, pl.ds(i, L)] = _xb[r, pl.ds(i, L)] + peb[pr, pl.ds(i, L)]

            return carry

        lax.fori_loop(0, CH_ROWS, row_body, 0)

        out_h[t] = start_out(t)
        nt = t + AHEAD
        if nt < NIT:
            prev = nt - DEPTH
            if prev >= 0:
                out_h[prev].wait()
            in_h[nt] = start_in(nt)

    for t in range(max(0, NIT - DEPTH), NIT):
        if t in out_h:
            out_h[t].wait()


@jax.jit
def _sc_call(x, pe):
    mesh = plsc.VectorSubcoreMesh(core_axis_name="c", subcore_axis_name="s")
    return pl.kernel(
        _sc_body,
        out_type=jax.ShapeDtypeStruct((BATCH, SEQ, D), jnp.float32),
        mesh=mesh,
        scratch_types=(
            [pltpu.VMEM((S_PER_W, D), jnp.float32)]
            + [pltpu.VMEM((CH_ROWS, D), jnp.float32) for _ in range(DEPTH)]
            + [pltpu.SemaphoreType.DMA for _ in range(1 + 2 * DEPTH)]
        ),
    )(x, pe)


def kernel(x, pos_table):
    batch, seq_len, d_model = x.shape
    pe = pos_table[:seq_len]
    return _sc_call(x, pe)
